# baseline (speedup 1.0000x reference)
import functools

import jax
import jax.numpy as jnp
import numpy as np
from jax import lax
from jax.experimental import pallas as pl
from jax.experimental.pallas import tpu as pltpu
from jax.experimental.pallas import tpu_sc as plsc

SPATIAL_SHAPES = ((64, 64), (32, 32), (16, 16), (8, 8))
EMBED_DIM = 256
NUM_HEADS = 8
NUM_LEVELS = 4
NUM_POINTS = 4
BS = 4
LEN_Q = 1024
LEN_V = sum(h * w for h, w in SPATIAL_SHAPES)
NEMBED = EMBED_DIM // NUM_HEADS
LP = NUM_LEVELS * NUM_POINTS
BH = BS * NUM_HEADS
TERMS = LP * 4
QCHUNK = 32
NCHUNK = LEN_Q // QCHUNK
ROWS_PER_CHUNK = QCHUNK * TERMS
IDX_ROWS = ROWS_PER_CHUNK // 128

_LEVEL_STARTS = []
_s = 0
for _h, _w in SPATIAL_SHAPES:
    _LEVEL_STARTS.append(_s)
    _s += _h * _w


def _lp_const(per_level_vals, dtype):
    v = np.zeros((NUM_LEVELS, NUM_POINTS), dtype=dtype)
    for li, val in enumerate(per_level_vals):
        v[li, :] = val
    return jnp.asarray(v.reshape(1, LP))


def _level_selector():
    s = np.zeros((NUM_LEVELS, LP), dtype=np.float32)
    for li in range(NUM_LEVELS):
        s[li, li * NUM_POINTS:(li + 1) * NUM_POINTS] = 1.0
    return jnp.asarray(s)


def _vproj_body(v_ref, w_ref, b_ref, m_ref, o_ref):
    v = v_ref[...]
    out = jnp.dot(v, w_ref[...], preferred_element_type=jnp.float32, precision=lax.Precision.HIGHEST) + b_ref[...]
    o_ref[...] = out * m_ref[...]


def _value_projection(value, vpk, vpb, maskf):
    return pl.pallas_call(
        _vproj_body,
        grid=(BS,),
        in_specs=[
            pl.BlockSpec((None, LEN_V, EMBED_DIM), lambda b: (b, 0, 0)),
            pl.BlockSpec((EMBED_DIM, EMBED_DIM), lambda b: (0, 0)),
            pl.BlockSpec((1, EMBED_DIM), lambda b: (0, 0)),
            pl.BlockSpec((None, LEN_V, 1), lambda b: (b, 0, 0)),
        ],
        out_specs=pl.BlockSpec((None, LEN_V, EMBED_DIM), lambda b: (b, 0, 0)),
        out_shape=jax.ShapeDtypeStruct((BS, LEN_V, EMBED_DIM), jnp.float32),
    )(value, vpk, vpb, maskf)


def _addr_body(q_ref, rx4_ref, ry4_ref, sel_ref, skx_ref, sky_ref, sbx_ref,
               sby_ref, ak_ref, ab_ref, wf_ref, hf_ref, wi_ref, hi_ref,
               st_ref, idx_ref, w_ref):
    b = pl.program_id(0)
    h = pl.program_id(1)
    q = q_ref[...]
    sel = sel_ref[...]
    refx = jnp.dot(rx4_ref[...], sel, preferred_element_type=jnp.float32, precision=lax.Precision.HIGHEST)
    refy = jnp.dot(ry4_ref[...], sel, preferred_element_type=jnp.float32, precision=lax.Precision.HIGHEST)
    offx = jnp.dot(q, skx_ref[...], preferred_element_type=jnp.float32, precision=lax.Precision.HIGHEST) + sbx_ref[...]
    offy = jnp.dot(q, sky_ref[...], preferred_element_type=jnp.float32, precision=lax.Precision.HIGHEST) + sby_ref[...]
    wf = wf_ref[...]
    hf = hf_ref[...]
    px = refx * wf + offx - 0.5
    py = refy * hf + offy - 0.5
    x0f = jnp.floor(px)
    y0f = jnp.floor(py)
    fx = px - x0f
    fy = py - y0f
    x0 = x0f.astype(jnp.int32)
    y0 = y0f.astype(jnp.int32)
    x1 = x0 + 1
    y1 = y0 + 1
    wi = wi_ref[...]
    hi = hi_ref[...]
    zero_i = jnp.zeros_like(x0)
    x0in = (x0 >= 0) & (x0 <= wi - 1)
    x1in = (x1 >= 0) & (x1 <= wi - 1)
    y0in = (y0 >= 0) & (y0 <= hi - 1)
    y1in = (y1 >= 0) & (y1 <= hi - 1)
    cx0 = jnp.clip(x0, zero_i, wi - 1)
    cx1 = jnp.clip(x1, zero_i, wi - 1)
    cy0 = jnp.clip(y0, zero_i, hi - 1)
    cy1 = jnp.clip(y1, zero_i, hi - 1)
    base = st_ref[...] + b * LEN_V
    ia = (cy0 * wi + cx0 + base) * NUM_HEADS + h
    ib = (cy1 * wi + cx0 + base) * NUM_HEADS + h
    ic = (cy0 * wi + cx1 + base) * NUM_HEADS + h
    idd = (cy1 * wi + cx1 + base) * NUM_HEADS + h
    s = jnp.dot(q, ak_ref[...], preferred_element_type=jnp.float32, precision=lax.Precision.HIGHEST) + ab_ref[...]
    m = jnp.max(s, axis=-1, keepdims=True)
    e = jnp.exp(s - m)
    a = e / jnp.sum(e, axis=-1, keepdims=True)
    onefx = 1.0 - fx
    onefy = 1.0 - fy
    zero_f = jnp.zeros_like(fx)
    wa = jnp.where(x0in & y0in, onefx * onefy, zero_f) * a
    wb = jnp.where(x0in & y1in, onefx * fy, zero_f) * a
    wc = jnp.where(x1in & y0in, fx * onefy, zero_f) * a
    wd = jnp.where(x1in & y1in, fx * fy, zero_f) * a
    idx_ref[...] = jnp.concatenate([ia, ib, ic, idd], axis=-1)
    w_ref[...] = jnp.concatenate([wa, wb, wc, wd], axis=-1)


def _addresses(query, rx4, ry4, skx, sky, sbx, sby, ak, ab):
    sel = _level_selector()
    wf = _lp_const([w for (_h, w) in SPATIAL_SHAPES], np.float32)
    hf = _lp_const([h for (h, _w) in SPATIAL_SHAPES], np.float32)
    wi = _lp_const([w for (_h, w) in SPATIAL_SHAPES], np.int32)
    hi = _lp_const([h for (h, _w) in SPATIAL_SHAPES], np.int32)
    st = _lp_const(_LEVEL_STARTS, np.int32)
    full = lambda shape: pl.BlockSpec(shape, lambda b, h: tuple(0 for _ in shape))
    hsl = lambda shape: pl.BlockSpec(shape, lambda b, h: (h, 0, 0))
    out_spec = pl.BlockSpec((None, None, LEN_Q, TERMS), lambda b, h: (b, h, 0, 0))
    return pl.pallas_call(
        _addr_body,
        grid=(BS, NUM_HEADS),
        in_specs=[
            pl.BlockSpec((None, LEN_Q, EMBED_DIM), lambda b, h: (b, 0, 0)),
            pl.BlockSpec((None, LEN_Q, NUM_LEVELS), lambda b, h: (b, 0, 0)),
            pl.BlockSpec((None, LEN_Q, NUM_LEVELS), lambda b, h: (b, 0, 0)),
            full((NUM_LEVELS, LP)),
            pl.BlockSpec((None, EMBED_DIM, LP), lambda b, h: (h, 0, 0)),
            pl.BlockSpec((None, EMBED_DIM, LP), lambda b, h: (h, 0, 0)),
            hsl((None, 1, LP)), hsl((None, 1, LP)),
            pl.BlockSpec((None, EMBED_DIM, LP), lambda b, h: (h, 0, 0)),
            hsl((None, 1, LP)),
            full((1, LP)), full((1, LP)), full((1, LP)), full((1, LP)),
            full((1, LP)),
        ],
        out_specs=[out_spec, out_spec],
        out_shape=[
            jax.ShapeDtypeStruct((BS, NUM_HEADS, LEN_Q, TERMS), jnp.int32),
            jax.ShapeDtypeStruct((BS, NUM_HEADS, LEN_Q, TERMS), jnp.float32),
        ],
    )(query, rx4, ry4, sel, skx, sky, sbx, sby, ak, ab, wf, hf, wi, hi, st)


def _sc_body(vflat_hbm, idx_hbm, w_hbm, out_hbm, idx_v, rows_v, w_v, out_v, sem):
    wid = lax.axis_index("s") * 2 + lax.axis_index("c")
    b = wid // NUM_HEADS
    h = wid % NUM_HEADS

    def chunk_body(cc, carry):
        pltpu.sync_copy(idx_hbm.at[wid, cc], idx_v)
        pltpu.sync_copy(w_hbm.at[wid, cc], w_v)
        copies = [
            pltpu.async_copy(vflat_hbm.at[idx_v.at[j]], rows_v.at[j], sem)
            for j in range(IDX_ROWS)
        ]
        for cp in copies:
            cp.wait()

        def q_body(qi, c2):
            jq = qi // 2
            rbase = (qi % 2) * TERMS
            fbase = qi * TERMS
            acc0 = jnp.zeros((16,), jnp.float32)
            acc1 = jnp.zeros((16,), jnp.float32)
            for g in range(TERMS // 16):
                wvec = w_v[pl.ds(fbase + g * 16, 16)]
                for k in range(16):
                    t = g * 16 + k
                    wt = wvec[k]
                    r = rbase + t
                    acc0 = acc0 + wt * rows_v[jq, r, pl.ds(0, 16)]
                    acc1 = acc1 + wt * rows_v[jq, r, pl.ds(16, 16)]
            out_v[qi, pl.ds(0, 16)] = acc0
            out_v[qi, pl.ds(16, 16)] = acc1
            return c2

        lax.fori_loop(0, QCHUNK, q_body, 0)
        pltpu.sync_copy(out_v, out_hbm.at[b, pl.ds(cc * QCHUNK, QCHUNK), h])
        return carry

    lax.fori_loop(0, NCHUNK, chunk_body, 0)


@functools.cache
def _sc_gather():
    return pl.kernel(
        _sc_body,
        mesh=plsc.VectorSubcoreMesh(core_axis_name="c", subcore_axis_name="s"),
        out_type=jax.ShapeDtypeStruct((BS, LEN_Q, NUM_HEADS, NEMBED), jnp.float32),
        compiler_params=pltpu.CompilerParams(use_tc_tiling_on_sc=False),
        scratch_types=[
            pltpu.VMEM((IDX_ROWS, 128), jnp.int32),
            pltpu.VMEM((IDX_ROWS, 128, NEMBED), jnp.float32),
            pltpu.VMEM((ROWS_PER_CHUNK,), jnp.float32),
            pltpu.VMEM((QCHUNK, NEMBED), jnp.float32),
            pltpu.SemaphoreType.DMA,
        ],
    )


def _oproj_body(x_ref, w_ref, b_ref, o_ref):
    o_ref[...] = (
        jnp.dot(x_ref[...], w_ref[...], preferred_element_type=jnp.float32, precision=lax.Precision.HIGHEST)
        + b_ref[...]
    )


def _out_projection(x, ok, ob):
    return pl.pallas_call(
        _oproj_body,
        grid=(BS,),
        in_specs=[
            pl.BlockSpec((None, LEN_Q, EMBED_DIM), lambda b: (b, 0, 0)),
            pl.BlockSpec((EMBED_DIM, EMBED_DIM), lambda b: (0, 0)),
            pl.BlockSpec((1, EMBED_DIM), lambda b: (0, 0)),
        ],
        out_specs=pl.BlockSpec((None, LEN_Q, EMBED_DIM), lambda b: (b, 0, 0)),
        out_shape=jax.ShapeDtypeStruct((BS, LEN_Q, EMBED_DIM), jnp.float32),
    )(x, ok, ob)


def kernel(query, ref_points, value, pad_mask, train, value_proj_kernel,
           value_proj_bias, samp_kernel, samp_bias, attn_kernel, attn_bias,
           out_kernel, out_bias):
    del train
    vpk = value_proj_kernel.reshape(EMBED_DIM, EMBED_DIM)
    vpb = value_proj_bias.reshape(1, EMBED_DIM)
    maskf = pad_mask.astype(jnp.float32).reshape(BS, LEN_V, 1)
    sk = samp_kernel.reshape(EMBED_DIM, NUM_HEADS, LP, 2)
    skx = sk[..., 0].transpose(1, 0, 2)
    sky = sk[..., 1].transpose(1, 0, 2)
    sb = samp_bias.reshape(NUM_HEADS, 1, LP, 2)
    sbx = sb[..., 0]
    sby = sb[..., 1]
    ak = attn_kernel.reshape(EMBED_DIM, NUM_HEADS, LP).transpose(1, 0, 2)
    ab = attn_bias.reshape(NUM_HEADS, 1, LP)
    rx4 = ref_points[..., 0]
    ry4 = ref_points[..., 1]

    vproj = _value_projection(value, vpk, vpb, maskf)
    vflat = vproj.reshape(BS * LEN_V * NUM_HEADS, NEMBED)

    idx_out, w_out = _addresses(query, rx4, ry4, skx, sky, sbx, sby, ak, ab)
    idxs = idx_out.reshape(BH, NCHUNK, IDX_ROWS, 128)
    ws = w_out.reshape(BH, NCHUNK, ROWS_PER_CHUNK)

    sc_out = _sc_gather()(vflat, idxs, ws)

    comb = sc_out.reshape(BS, LEN_Q, EMBED_DIM)
    return _out_projection(comb, out_kernel.reshape(EMBED_DIM, EMBED_DIM),
                           out_bias.reshape(1, EMBED_DIM))


# speedup vs baseline: 1.0149x; 1.0149x over previous
import functools

import jax
import jax.numpy as jnp
import numpy as np
from jax import lax
from jax.experimental import pallas as pl
from jax.experimental.pallas import tpu as pltpu
from jax.experimental.pallas import tpu_sc as plsc

SPATIAL_SHAPES = ((64, 64), (32, 32), (16, 16), (8, 8))
EMBED_DIM = 256
NUM_HEADS = 8
NUM_LEVELS = 4
NUM_POINTS = 4
BS = 4
LEN_Q = 1024
LEN_V = sum(h * w for h, w in SPATIAL_SHAPES)
NEMBED = EMBED_DIM // NUM_HEADS
LP = NUM_LEVELS * NUM_POINTS
BH = BS * NUM_HEADS
TERMS = LP * 4
QCHUNK = 32
NCHUNK = LEN_Q // QCHUNK
ROWS_PER_CHUNK = QCHUNK * TERMS
IDX_ROWS = ROWS_PER_CHUNK // 128

_LEVEL_STARTS = []
_s = 0
for _h, _w in SPATIAL_SHAPES:
    _LEVEL_STARTS.append(_s)
    _s += _h * _w


def _lp_const(per_level_vals, dtype):
    v = np.zeros((NUM_LEVELS, NUM_POINTS), dtype=dtype)
    for li, val in enumerate(per_level_vals):
        v[li, :] = val
    return jnp.asarray(v.reshape(1, LP))


def _level_selector():
    s = np.zeros((NUM_LEVELS, LP), dtype=np.float32)
    for li in range(NUM_LEVELS):
        s[li, li * NUM_POINTS:(li + 1) * NUM_POINTS] = 1.0
    return jnp.asarray(s)


def _vproj_body(v_ref, w_ref, b_ref, m_ref, o_ref):
    v = v_ref[...]
    out = jnp.dot(v, w_ref[...], preferred_element_type=jnp.float32) + b_ref[...]
    o_ref[...] = out * m_ref[...]


def _value_projection(value, vpk, vpb, maskf):
    return pl.pallas_call(
        _vproj_body,
        grid=(BS,),
        in_specs=[
            pl.BlockSpec((None, LEN_V, EMBED_DIM), lambda b: (b, 0, 0)),
            pl.BlockSpec((EMBED_DIM, EMBED_DIM), lambda b: (0, 0)),
            pl.BlockSpec((1, EMBED_DIM), lambda b: (0, 0)),
            pl.BlockSpec((None, LEN_V, 1), lambda b: (b, 0, 0)),
        ],
        out_specs=pl.BlockSpec((None, LEN_V, EMBED_DIM), lambda b: (b, 0, 0)),
        out_shape=jax.ShapeDtypeStruct((BS, LEN_V, EMBED_DIM), jnp.float32),
    )(value, vpk, vpb, maskf)


def _addr_body(q_ref, rx4_ref, ry4_ref, sel_ref, skx_ref, sky_ref, sbx_ref,
               sby_ref, ak_ref, ab_ref, wf_ref, hf_ref, wi_ref, hi_ref,
               st_ref, idx_ref, w_ref):
    b = pl.program_id(0)
    h = pl.program_id(1)
    q = q_ref[...]
    sel = sel_ref[...]
    refx = jnp.dot(rx4_ref[...], sel, preferred_element_type=jnp.float32, precision=lax.Precision.HIGHEST)
    refy = jnp.dot(ry4_ref[...], sel, preferred_element_type=jnp.float32, precision=lax.Precision.HIGHEST)
    offx = jnp.dot(q, skx_ref[...], preferred_element_type=jnp.float32, precision=lax.Precision.HIGHEST) + sbx_ref[...]
    offy = jnp.dot(q, sky_ref[...], preferred_element_type=jnp.float32, precision=lax.Precision.HIGHEST) + sby_ref[...]
    wf = wf_ref[...]
    hf = hf_ref[...]
    px = refx * wf + offx - 0.5
    py = refy * hf + offy - 0.5
    x0f = jnp.floor(px)
    y0f = jnp.floor(py)
    fx = px - x0f
    fy = py - y0f
    x0 = x0f.astype(jnp.int32)
    y0 = y0f.astype(jnp.int32)
    x1 = x0 + 1
    y1 = y0 + 1
    wi = wi_ref[...]
    hi = hi_ref[...]
    zero_i = jnp.zeros_like(x0)
    x0in = (x0 >= 0) & (x0 <= wi - 1)
    x1in = (x1 >= 0) & (x1 <= wi - 1)
    y0in = (y0 >= 0) & (y0 <= hi - 1)
    y1in = (y1 >= 0) & (y1 <= hi - 1)
    cx0 = jnp.clip(x0, zero_i, wi - 1)
    cx1 = jnp.clip(x1, zero_i, wi - 1)
    cy0 = jnp.clip(y0, zero_i, hi - 1)
    cy1 = jnp.clip(y1, zero_i, hi - 1)
    base = st_ref[...] + b * LEN_V
    ia = (cy0 * wi + cx0 + base) * NUM_HEADS + h
    ib = (cy1 * wi + cx0 + base) * NUM_HEADS + h
    ic = (cy0 * wi + cx1 + base) * NUM_HEADS + h
    idd = (cy1 * wi + cx1 + base) * NUM_HEADS + h
    s = jnp.dot(q, ak_ref[...], preferred_element_type=jnp.float32, precision=lax.Precision.HIGHEST) + ab_ref[...]
    m = jnp.max(s, axis=-1, keepdims=True)
    e = jnp.exp(s - m)
    a = e / jnp.sum(e, axis=-1, keepdims=True)
    onefx = 1.0 - fx
    onefy = 1.0 - fy
    zero_f = jnp.zeros_like(fx)
    wa = jnp.where(x0in & y0in, onefx * onefy, zero_f) * a
    wb = jnp.where(x0in & y1in, onefx * fy, zero_f) * a
    wc = jnp.where(x1in & y0in, fx * onefy, zero_f) * a
    wd = jnp.where(x1in & y1in, fx * fy, zero_f) * a
    idx_ref[...] = jnp.concatenate([ia, ib, ic, idd], axis=-1)
    w_ref[...] = jnp.concatenate([wa, wb, wc, wd], axis=-1)


def _addresses(query, rx4, ry4, skx, sky, sbx, sby, ak, ab):
    sel = _level_selector()
    wf = _lp_const([w for (_h, w) in SPATIAL_SHAPES], np.float32)
    hf = _lp_const([h for (h, _w) in SPATIAL_SHAPES], np.float32)
    wi = _lp_const([w for (_h, w) in SPATIAL_SHAPES], np.int32)
    hi = _lp_const([h for (h, _w) in SPATIAL_SHAPES], np.int32)
    st = _lp_const(_LEVEL_STARTS, np.int32)
    full = lambda shape: pl.BlockSpec(shape, lambda b, h: tuple(0 for _ in shape))
    hsl = lambda shape: pl.BlockSpec(shape, lambda b, h: (h, 0, 0))
    out_spec = pl.BlockSpec((None, None, LEN_Q, TERMS), lambda b, h: (b, h, 0, 0))
    return pl.pallas_call(
        _addr_body,
        grid=(BS, NUM_HEADS),
        in_specs=[
            pl.BlockSpec((None, LEN_Q, EMBED_DIM), lambda b, h: (b, 0, 0)),
            pl.BlockSpec((None, LEN_Q, NUM_LEVELS), lambda b, h: (b, 0, 0)),
            pl.BlockSpec((None, LEN_Q, NUM_LEVELS), lambda b, h: (b, 0, 0)),
            full((NUM_LEVELS, LP)),
            pl.BlockSpec((None, EMBED_DIM, LP), lambda b, h: (h, 0, 0)),
            pl.BlockSpec((None, EMBED_DIM, LP), lambda b, h: (h, 0, 0)),
            hsl((None, 1, LP)), hsl((None, 1, LP)),
            pl.BlockSpec((None, EMBED_DIM, LP), lambda b, h: (h, 0, 0)),
            hsl((None, 1, LP)),
            full((1, LP)), full((1, LP)), full((1, LP)), full((1, LP)),
            full((1, LP)),
        ],
        out_specs=[out_spec, out_spec],
        out_shape=[
            jax.ShapeDtypeStruct((BS, NUM_HEADS, LEN_Q, TERMS), jnp.int32),
            jax.ShapeDtypeStruct((BS, NUM_HEADS, LEN_Q, TERMS), jnp.float32),
        ],
    )(query, rx4, ry4, sel, skx, sky, sbx, sby, ak, ab, wf, hf, wi, hi, st)


def _sc_body(vflat_hbm, idx_hbm, w_hbm, out_hbm, idx_v, rows_v, w_v, out_v, sem):
    wid = lax.axis_index("s") * 2 + lax.axis_index("c")
    b = wid // NUM_HEADS
    h = wid % NUM_HEADS

    def chunk_body(cc, carry):
        pltpu.sync_copy(idx_hbm.at[wid, cc], idx_v)
        pltpu.sync_copy(w_hbm.at[wid, cc], w_v)
        copies = [
            pltpu.async_copy(vflat_hbm.at[idx_v.at[j]], rows_v.at[j], sem)
            for j in range(IDX_ROWS)
        ]
        for cp in copies:
            cp.wait()

        def q_body(qi, c2):
            jq = qi // 2
            rbase = (qi % 2) * TERMS
            fbase = qi * TERMS
            acc0 = jnp.zeros((16,), jnp.float32)
            acc1 = jnp.zeros((16,), jnp.float32)
            for g in range(TERMS // 16):
                wvec = w_v[pl.ds(fbase + g * 16, 16)]
                for k in range(16):
                    t = g * 16 + k
                    wt = wvec[k]
                    r = rbase + t
                    acc0 = acc0 + wt * rows_v[jq, r, pl.ds(0, 16)]
                    acc1 = acc1 + wt * rows_v[jq, r, pl.ds(16, 16)]
            out_v[qi, pl.ds(0, 16)] = acc0
            out_v[qi, pl.ds(16, 16)] = acc1
            return c2

        lax.fori_loop(0, QCHUNK, q_body, 0)
        pltpu.sync_copy(out_v, out_hbm.at[b, pl.ds(cc * QCHUNK, QCHUNK), h])
        return carry

    lax.fori_loop(0, NCHUNK, chunk_body, 0)


@functools.cache
def _sc_gather():
    return pl.kernel(
        _sc_body,
        mesh=plsc.VectorSubcoreMesh(core_axis_name="c", subcore_axis_name="s"),
        out_type=jax.ShapeDtypeStruct((BS, LEN_Q, NUM_HEADS, NEMBED), jnp.float32),
        compiler_params=pltpu.CompilerParams(use_tc_tiling_on_sc=False),
        scratch_types=[
            pltpu.VMEM((IDX_ROWS, 128), jnp.int32),
            pltpu.VMEM((IDX_ROWS, 128, NEMBED), jnp.float32),
            pltpu.VMEM((ROWS_PER_CHUNK,), jnp.float32),
            pltpu.VMEM((QCHUNK, NEMBED), jnp.float32),
            pltpu.SemaphoreType.DMA,
        ],
    )


def _oproj_body(x_ref, w_ref, b_ref, o_ref):
    o_ref[...] = (
        jnp.dot(x_ref[...], w_ref[...], preferred_element_type=jnp.float32)
        + b_ref[...]
    )


def _out_projection(x, ok, ob):
    return pl.pallas_call(
        _oproj_body,
        grid=(BS,),
        in_specs=[
            pl.BlockSpec((None, LEN_Q, EMBED_DIM), lambda b: (b, 0, 0)),
            pl.BlockSpec((EMBED_DIM, EMBED_DIM), lambda b: (0, 0)),
            pl.BlockSpec((1, EMBED_DIM), lambda b: (0, 0)),
        ],
        out_specs=pl.BlockSpec((None, LEN_Q, EMBED_DIM), lambda b: (b, 0, 0)),
        out_shape=jax.ShapeDtypeStruct((BS, LEN_Q, EMBED_DIM), jnp.float32),
    )(x, ok, ob)


def kernel(query, ref_points, value, pad_mask, train, value_proj_kernel,
           value_proj_bias, samp_kernel, samp_bias, attn_kernel, attn_bias,
           out_kernel, out_bias):
    del train
    vpk = value_proj_kernel.reshape(EMBED_DIM, EMBED_DIM)
    vpb = value_proj_bias.reshape(1, EMBED_DIM)
    maskf = pad_mask.astype(jnp.float32).reshape(BS, LEN_V, 1)
    sk = samp_kernel.reshape(EMBED_DIM, NUM_HEADS, LP, 2)
    skx = sk[..., 0].transpose(1, 0, 2)
    sky = sk[..., 1].transpose(1, 0, 2)
    sb = samp_bias.reshape(NUM_HEADS, 1, LP, 2)
    sbx = sb[..., 0]
    sby = sb[..., 1]
    ak = attn_kernel.reshape(EMBED_DIM, NUM_HEADS, LP).transpose(1, 0, 2)
    ab = attn_bias.reshape(NUM_HEADS, 1, LP)
    rx4 = ref_points[..., 0]
    ry4 = ref_points[..., 1]

    vproj = _value_projection(value, vpk, vpb, maskf)
    vflat = vproj.reshape(BS * LEN_V * NUM_HEADS, NEMBED)

    idx_out, w_out = _addresses(query, rx4, ry4, skx, sky, sbx, sby, ak, ab)
    idxs = idx_out.reshape(BH, NCHUNK, IDX_ROWS, 128)
    ws = w_out.reshape(BH, NCHUNK, ROWS_PER_CHUNK)

    sc_out = _sc_gather()(vflat, idxs, ws)

    comb = sc_out.reshape(BS, LEN_Q, EMBED_DIM)
    return _out_projection(comb, out_kernel.reshape(EMBED_DIM, EMBED_DIM),
                           out_bias.reshape(1, EMBED_DIM))


# speedup vs baseline: 1.2155x; 1.1976x over previous
import functools

import jax
import jax.numpy as jnp
import numpy as np
from jax import lax
from jax.experimental import pallas as pl
from jax.experimental.pallas import tpu as pltpu
from jax.experimental.pallas import tpu_sc as plsc

SPATIAL_SHAPES = ((64, 64), (32, 32), (16, 16), (8, 8))
EMBED_DIM = 256
NUM_HEADS = 8
NUM_LEVELS = 4
NUM_POINTS = 4
BS = 4
LEN_Q = 1024
LEN_V = sum(h * w for h, w in SPATIAL_SHAPES)
NEMBED = EMBED_DIM // NUM_HEADS
LP = NUM_LEVELS * NUM_POINTS
BH = BS * NUM_HEADS
TERMS = LP * 4
QCHUNK = 32
NCHUNK = LEN_Q // QCHUNK
ROWS_PER_CHUNK = QCHUNK * TERMS
IDX_ROWS = ROWS_PER_CHUNK // 128

_LEVEL_STARTS = []
_s = 0
for _h, _w in SPATIAL_SHAPES:
    _LEVEL_STARTS.append(_s)
    _s += _h * _w


def _lp_const(per_level_vals, dtype):
    v = np.zeros((NUM_LEVELS, NUM_POINTS), dtype=dtype)
    for li, val in enumerate(per_level_vals):
        v[li, :] = val
    return jnp.asarray(v.reshape(1, LP))


def _level_selector():
    s = np.zeros((NUM_LEVELS, LP), dtype=np.float32)
    for li in range(NUM_LEVELS):
        s[li, li * NUM_POINTS:(li + 1) * NUM_POINTS] = 1.0
    return jnp.asarray(s)


def _vproj_body(v_ref, w_ref, b_ref, m_ref, o_ref):
    v = v_ref[...]
    out = jnp.dot(v, w_ref[...], preferred_element_type=jnp.float32) + b_ref[...]
    o_ref[...] = out * m_ref[...]


def _value_projection(value, vpk, vpb, maskf):
    return pl.pallas_call(
        _vproj_body,
        grid=(BS,),
        in_specs=[
            pl.BlockSpec((None, LEN_V, EMBED_DIM), lambda b: (b, 0, 0)),
            pl.BlockSpec((EMBED_DIM, EMBED_DIM), lambda b: (0, 0)),
            pl.BlockSpec((1, EMBED_DIM), lambda b: (0, 0)),
            pl.BlockSpec((None, LEN_V, 1), lambda b: (b, 0, 0)),
        ],
        out_specs=pl.BlockSpec((None, LEN_V, EMBED_DIM), lambda b: (b, 0, 0)),
        out_shape=jax.ShapeDtypeStruct((BS, LEN_V, EMBED_DIM), jnp.float32),
    )(value, vpk, vpb, maskf)


def _addr_body(q_ref, refx_ref, refy_ref, skk_ref, b48_ref, wf_ref, hf_ref,
               wi_ref, hi_ref, st_ref, idx_ref, w_ref):
    b = pl.program_id(0)
    h = pl.program_id(1)
    q = q_ref[...]
    o = jnp.dot(q, skk_ref[...], preferred_element_type=jnp.float32,
                precision=lax.Precision.HIGHEST) + b48_ref[...]
    offx = o[:, 0:LP]
    offy = o[:, LP:2 * LP]
    s = o[:, 2 * LP:3 * LP]
    refx = refx_ref[...]
    refy = refy_ref[...]
    wf = wf_ref[...]
    hf = hf_ref[...]
    px = refx * wf + offx - 0.5
    py = refy * hf + offy - 0.5
    x0f = jnp.floor(px)
    y0f = jnp.floor(py)
    fx = px - x0f
    fy = py - y0f
    x0 = x0f.astype(jnp.int32)
    y0 = y0f.astype(jnp.int32)
    x1 = x0 + 1
    y1 = y0 + 1
    wi = wi_ref[...]
    hi = hi_ref[...]
    zero_i = jnp.zeros_like(x0)
    x0in = (x0 >= 0) & (x0 <= wi - 1)
    x1in = (x1 >= 0) & (x1 <= wi - 1)
    y0in = (y0 >= 0) & (y0 <= hi - 1)
    y1in = (y1 >= 0) & (y1 <= hi - 1)
    cx0 = jnp.clip(x0, zero_i, wi - 1)
    cx1 = jnp.clip(x1, zero_i, wi - 1)
    cy0 = jnp.clip(y0, zero_i, hi - 1)
    cy1 = jnp.clip(y1, zero_i, hi - 1)
    base = st_ref[...] + b * LEN_V
    ia = (cy0 * wi + cx0 + base) * NUM_HEADS + h
    ib = (cy1 * wi + cx0 + base) * NUM_HEADS + h
    ic = (cy0 * wi + cx1 + base) * NUM_HEADS + h
    idd = (cy1 * wi + cx1 + base) * NUM_HEADS + h
    m = jnp.max(s, axis=-1, keepdims=True)
    e = jnp.exp(s - m)
    a = e / jnp.sum(e, axis=-1, keepdims=True)
    onefx = 1.0 - fx
    onefy = 1.0 - fy
    zero_f = jnp.zeros_like(fx)
    wa = jnp.where(x0in & y0in, onefx * onefy, zero_f) * a
    wb = jnp.where(x0in & y1in, onefx * fy, zero_f) * a
    wc = jnp.where(x1in & y0in, fx * onefy, zero_f) * a
    wd = jnp.where(x1in & y1in, fx * fy, zero_f) * a
    idx_ref[...] = jnp.concatenate([ia, ib, ic, idd], axis=-1)
    w_ref[...] = jnp.concatenate([wa, wb, wc, wd], axis=-1)


def _addresses(query, refx, refy, skk, b48):
    wf = _lp_const([w for (_h, w) in SPATIAL_SHAPES], np.float32)
    hf = _lp_const([h for (h, _w) in SPATIAL_SHAPES], np.float32)
    wi = _lp_const([w for (_h, w) in SPATIAL_SHAPES], np.int32)
    hi = _lp_const([h for (h, _w) in SPATIAL_SHAPES], np.int32)
    st = _lp_const(_LEVEL_STARTS, np.int32)
    full = lambda shape: pl.BlockSpec(shape, lambda b, h: tuple(0 for _ in shape))
    hsl = lambda shape: pl.BlockSpec(shape, lambda b, h: (h, 0, 0))
    out_spec = pl.BlockSpec((None, None, LEN_Q, TERMS), lambda b, h: (b, h, 0, 0))
    return pl.pallas_call(
        _addr_body,
        grid=(BS, NUM_HEADS),
        in_specs=[
            pl.BlockSpec((None, LEN_Q, EMBED_DIM), lambda b, h: (b, 0, 0)),
            pl.BlockSpec((None, LEN_Q, LP), lambda b, h: (b, 0, 0)),
            pl.BlockSpec((None, LEN_Q, LP), lambda b, h: (b, 0, 0)),
            pl.BlockSpec((None, EMBED_DIM, 3 * LP), lambda b, h: (h, 0, 0)),
            hsl((None, 1, 3 * LP)),
            full((1, LP)), full((1, LP)), full((1, LP)), full((1, LP)),
            full((1, LP)),
        ],
        out_specs=[out_spec, out_spec],
        out_shape=[
            jax.ShapeDtypeStruct((BS, NUM_HEADS, LEN_Q, TERMS), jnp.int32),
            jax.ShapeDtypeStruct((BS, NUM_HEADS, LEN_Q, TERMS), jnp.float32),
        ],
    )(query, refx, refy, skk, b48, wf, hf, wi, hi, st)


def _sc_body(vflat_hbm, idx_hbm, w_hbm, out_hbm, idx_v, rows_v, w_v, out_v, sem):
    wid = lax.axis_index("s") * 2 + lax.axis_index("c")
    b = wid // NUM_HEADS
    h = wid % NUM_HEADS

    def chunk_body(cc, carry):
        pltpu.sync_copy(idx_hbm.at[wid, cc], idx_v)
        pltpu.sync_copy(w_hbm.at[wid, cc], w_v)
        copies = [
            pltpu.async_copy(vflat_hbm.at[idx_v.at[j]], rows_v.at[j], sem)
            for j in range(IDX_ROWS)
        ]
        for cp in copies:
            cp.wait()

        def q_body(qi, c2):
            jq = qi // 2
            rbase = (qi % 2) * TERMS
            fbase = qi * TERMS
            acc0 = jnp.zeros((16,), jnp.float32)
            acc1 = jnp.zeros((16,), jnp.float32)
            for g in range(TERMS // 16):
                wvec = w_v[pl.ds(fbase + g * 16, 16)]
                for k in range(16):
                    t = g * 16 + k
                    wt = wvec[k]
                    r = rbase + t
                    acc0 = acc0 + wt * rows_v[jq, r, pl.ds(0, 16)]
                    acc1 = acc1 + wt * rows_v[jq, r, pl.ds(16, 16)]
            out_v[qi, pl.ds(0, 16)] = acc0
            out_v[qi, pl.ds(16, 16)] = acc1
            return c2

        lax.fori_loop(0, QCHUNK, q_body, 0)
        pltpu.sync_copy(out_v, out_hbm.at[b, pl.ds(cc * QCHUNK, QCHUNK), h])
        return carry

    lax.fori_loop(0, NCHUNK, chunk_body, 0)


@functools.cache
def _sc_gather():
    return pl.kernel(
        _sc_body,
        mesh=plsc.VectorSubcoreMesh(core_axis_name="c", subcore_axis_name="s"),
        out_type=jax.ShapeDtypeStruct((BS, LEN_Q, NUM_HEADS, NEMBED), jnp.float32),
        compiler_params=pltpu.CompilerParams(use_tc_tiling_on_sc=False),
        scratch_types=[
            pltpu.VMEM((IDX_ROWS, 128), jnp.int32),
            pltpu.VMEM((IDX_ROWS, 128, NEMBED), jnp.float32),
            pltpu.VMEM((ROWS_PER_CHUNK,), jnp.float32),
            pltpu.VMEM((QCHUNK, NEMBED), jnp.float32),
            pltpu.SemaphoreType.DMA,
        ],
    )


def _oproj_body(x_ref, w_ref, b_ref, o_ref):
    o_ref[...] = (
        jnp.dot(x_ref[...], w_ref[...], preferred_element_type=jnp.float32)
        + b_ref[...]
    )


def _out_projection(x, ok, ob):
    return pl.pallas_call(
        _oproj_body,
        grid=(BS,),
        in_specs=[
            pl.BlockSpec((None, LEN_Q, EMBED_DIM), lambda b: (b, 0, 0)),
            pl.BlockSpec((EMBED_DIM, EMBED_DIM), lambda b: (0, 0)),
            pl.BlockSpec((1, EMBED_DIM), lambda b: (0, 0)),
        ],
        out_specs=pl.BlockSpec((None, LEN_Q, EMBED_DIM), lambda b: (b, 0, 0)),
        out_shape=jax.ShapeDtypeStruct((BS, LEN_Q, EMBED_DIM), jnp.float32),
    )(x, ok, ob)


def kernel(query, ref_points, value, pad_mask, train, value_proj_kernel,
           value_proj_bias, samp_kernel, samp_bias, attn_kernel, attn_bias,
           out_kernel, out_bias):
    del train
    vpk = value_proj_kernel.reshape(EMBED_DIM, EMBED_DIM)
    vpb = value_proj_bias.reshape(1, EMBED_DIM)
    maskf = pad_mask.astype(jnp.float32).reshape(BS, LEN_V, 1)
    sk = samp_kernel.reshape(EMBED_DIM, NUM_HEADS, LP, 2)
    skx = sk[..., 0].transpose(1, 0, 2)
    sky = sk[..., 1].transpose(1, 0, 2)
    sb = samp_bias.reshape(NUM_HEADS, 1, LP, 2)
    ak = attn_kernel.reshape(EMBED_DIM, NUM_HEADS, LP).transpose(1, 0, 2)
    ab = attn_bias.reshape(NUM_HEADS, 1, LP)
    skk = jnp.concatenate([skx, sky, ak], axis=-1)
    b48 = jnp.concatenate([sb[..., 0], sb[..., 1], ab], axis=-1)
    refx = jnp.broadcast_to(
        ref_points[:, :, :, None, 0], (BS, LEN_Q, NUM_LEVELS, NUM_POINTS)
    ).reshape(BS, LEN_Q, LP)
    refy = jnp.broadcast_to(
        ref_points[:, :, :, None, 1], (BS, LEN_Q, NUM_LEVELS, NUM_POINTS)
    ).reshape(BS, LEN_Q, LP)

    vproj = _value_projection(value, vpk, vpb, maskf)
    vflat = vproj.reshape(BS * LEN_V * NUM_HEADS, NEMBED)

    idx_out, w_out = _addresses(query, refx, refy, skk, b48)
    idxs = idx_out.reshape(BH, NCHUNK, IDX_ROWS, 128)
    ws = w_out.reshape(BH, NCHUNK, ROWS_PER_CHUNK)

    sc_out = _sc_gather()(vflat, idxs, ws)

    comb = sc_out.reshape(BS, LEN_Q, EMBED_DIM)
    return _out_projection(comb, out_kernel.reshape(EMBED_DIM, EMBED_DIM),
                           out_bias.reshape(1, EMBED_DIM))
